# Initial kernel scaffold; baseline (speedup 1.0000x reference)
import jax
import jax.numpy as jnp
from jax.experimental import pallas as pl


def kernel(z, edge_index, D, params):
    raise NotImplementedError("write your pallas kernel here")


# speedup vs baseline: 284.2667x; 284.2667x over previous
import functools

import jax
import jax.numpy as jnp
from jax import lax
from jax.experimental import pallas as pl
from jax.experimental.pallas import tpu as pltpu
from jax.experimental.pallas import tpu_sc as plsc

_NC = 2
_NS = 16
_LANES = 16


@functools.partial(jax.jit, static_argnums=(3, 4, 5))
def _segment_partials(zp, src, dst, NPAD, E, W):
    NW = _NC * _NS
    EW = E // NW
    NWIN = EW // W
    assert EW * NW == E and NWIN * W == EW and W % 8 == 0
    SL = NPAD // _NS
    assert SL * _NS == NPAD and SL % 128 == 0

    mesh = plsc.VectorSubcoreMesh(core_axis_name="c", subcore_axis_name="s")

    @functools.partial(
        pl.kernel,
        out_type=jax.ShapeDtypeStruct((_NC, NPAD), jnp.float32),
        mesh=mesh,
        scratch_types=[
            pltpu.VMEM_SHARED((NPAD,), jnp.float32),
            pltpu.VMEM_SHARED((NPAD,), jnp.float32),
            pltpu.VMEM((W,), jnp.int32),
            pltpu.VMEM((W,), jnp.int32),
            pltpu.VMEM((W,), jnp.float32),
            pltpu.VMEM((SL,), jnp.float32),
        ],
    )
    def seg(z_hbm, src_hbm, dst_hbm, out_hbm,
            z_sh, acc_sh, dst_v, src_v, val_v, zero_v):
        cid = lax.axis_index("c")
        sid = lax.axis_index("s")

        def zbody(i, carry):
            zero_v[pl.ds(i * _LANES, _LANES)] = jnp.zeros((_LANES,), jnp.float32)
            return carry

        lax.fori_loop(0, SL // _LANES, zbody, 0)
        pltpu.sync_copy(zero_v, acc_sh.at[pl.ds(sid * SL, SL)])

        @pl.when(sid == 0)
        def _stage_z():
            pltpu.sync_copy(z_hbm, z_sh)

        plsc.subcore_barrier()

        ebase = (cid * _NS + sid) * EW

        def body(i, carry):
            off = ebase + i * W
            pltpu.sync_copy(dst_hbm.at[pl.ds(off, W)], dst_v)
            pltpu.sync_copy(src_hbm.at[pl.ds(off, W)], src_v)
            pltpu.sync_copy(z_sh.at[dst_v], val_v)
            pltpu.sync_copy(val_v, acc_sh.at[src_v], add=True)
            return carry

        lax.fori_loop(0, NWIN, body, 0)

        plsc.subcore_barrier()

        @pl.when(sid == 0)
        def _writeout():
            pltpu.sync_copy(acc_sh, out_hbm.at[cid])

    return seg(zp, src, dst)


def _combine_body(s_ref, z_ref, d_ref, p_ref, o_ref):
    alpha = s_ref[0]
    beta = s_ref[1]
    gamma = s_ref[2]
    bias = s_ref[3]
    logd = jnp.log(d_ref[...])
    az = p_ref[0] + p_ref[1]
    o_ref[...] = (alpha * jnp.exp(gamma * logd) * z_ref[...]
                  + beta * jnp.exp((gamma - 1.0) * logd) * az + bias)


def kernel(z, edge_index, D, params):
    N = z.shape[0]
    E = edge_index.shape[1]
    src = edge_index[0]
    dst = edge_index[1]

    NPAD = -(-N // (_NS * 128)) * (_NS * 128)
    zp = jnp.pad(z, (0, NPAD - N))
    dp = jnp.pad(D, (0, NPAD - N), constant_values=1.0)

    partial = _segment_partials(zp, src, dst, NPAD, E, 10000)

    alpha = jnp.exp(params[0])
    beta = -alpha * jnp.exp(params[1])
    gamma = jnp.exp(params[2])
    scal = jnp.stack([alpha, beta, gamma, params[3]])

    combine = pl.pallas_call(
        _combine_body,
        out_shape=jax.ShapeDtypeStruct((NPAD,), jnp.float32),
        in_specs=[
            pl.BlockSpec(memory_space=pltpu.SMEM),
            pl.BlockSpec(memory_space=pltpu.VMEM),
            pl.BlockSpec(memory_space=pltpu.VMEM),
            pl.BlockSpec(memory_space=pltpu.VMEM),
        ],
        out_specs=pl.BlockSpec(memory_space=pltpu.VMEM),
    )
    return combine(scal, zp, dp, partial)[:N]
